# baseline (speedup 1.0000x reference)
import functools

import jax
import jax.numpy as jnp
from jax import lax
from jax.experimental import pallas as pl
from jax.experimental.pallas import tpu as pltpu
from jax.experimental.pallas import tpu_sc as plsc

DIM = 64
SCALE = 8.0
LANES = 16
CHUNK = 256
MAXPOS = 200


VOCAB_BLOCK = 2048


@functools.lru_cache(maxsize=None)
def _linearize_table(vocab: int, dim: int):
    nb = (vocab + VOCAB_BLOCK - 1) // VOCAB_BLOCK

    def body(wt_ref, out_ref):
        ii = lax.broadcasted_iota(jnp.int32, (dim, dim), 0)
        jj = lax.broadcasted_iota(jnp.int32, (dim, dim), 1)
        eye = (ii == jj).astype(jnp.float32)
        t = jax.lax.dot_general(wt_ref[...], eye, (((0,), (0,)), ((), ())))
        t3 = t.reshape(VOCAB_BLOCK // 2, 2, dim)
        out_ref[...] = jnp.concatenate([t3[:, 0, :], t3[:, 1, :]], axis=1)

    return pl.pallas_call(
        body,
        grid=(nb,),
        in_specs=[pl.BlockSpec((dim, VOCAB_BLOCK), lambda i: (0, i))],
        out_specs=pl.BlockSpec((VOCAB_BLOCK // 2, 2 * dim), lambda i: (i, 0)),
        out_shape=jax.ShapeDtypeStruct((vocab // 2, 2 * dim), jnp.float32),
    )


@functools.lru_cache(maxsize=None)
def _build(n_rows: int):
    info = plsc.get_sparse_core_info()
    nw = info.num_cores * info.num_subcores
    rows_per_w = n_rows // nw
    n_chunks = rows_per_w // CHUNK
    assert rows_per_w % CHUNK == 0 and n_chunks >= 4 and (n_chunks - 4) % 4 == 0

    mesh = plsc.VectorSubcoreMesh(core_axis_name="c", subcore_axis_name="s")

    @functools.partial(
        pl.kernel,
        mesh=mesh,
        compiler_params=pltpu.CompilerParams(use_tc_tiling_on_sc=False),
        out_type=jax.ShapeDtypeStruct((n_rows, 2 * DIM), jnp.float32),
        scratch_types=(
            [pltpu.VMEM((rows_per_w,), jnp.int32)] * 2
            + [pltpu.VMEM((CHUNK, DIM), jnp.float32)] * 4
            + [pltpu.SemaphoreType.DMA] * 12
        ),
    )
    def emb(w_word, w_pos, iw_hbm, ip_hbm, out_hbm, iw_v, ip_v, *rest):
        wid = lax.axis_index("s") * info.num_cores + lax.axis_index("c")
        base = wid * rows_per_w
        pltpu.sync_copy(iw_hbm.at[pl.ds(base, rows_per_w)], iw_v)
        pltpu.sync_copy(ip_hbm.at[pl.ds(base, rows_per_w)], ip_v)

        bufv, semv = rest[:4], rest[4:]
        bufs = tuple(
            (bufv[i], semv[3 * i], semv[3 * i + 1], semv[3 * i + 2]) for i in range(4)
        )

        def issue_word(c, slot):
            bw, sw, _, _ = slot
            pltpu.async_copy(w_word.at[iw_v.at[pl.ds(c * CHUNK, CHUNK)]], bw, sw)

        def wait_word(slot):
            bw, sw, _, _ = slot
            pltpu.make_async_copy(w_word.at[iw_v.at[pl.ds(0, CHUNK)]], bw, sw).wait()

        def scale(slot):
            bw = slot[0]

            def row_body(r, carry):
                for k in range(4):
                    ri = r * 4 + k
                    for j in range(DIM // LANES):
                        s = pl.ds(j * LANES, LANES)
                        bw[ri, s] = bw[ri, s] * SCALE
                return carry

            lax.fori_loop(0, CHUNK // 4, row_body, 0, unroll=False)

        def issue_posadd(c, slot):
            bw, _, sp, _ = slot
            pltpu.async_copy(
                w_pos.at[ip_v.at[pl.ds(c * CHUNK, CHUNK)]], bw, sp, add=True
            )

        def wait_posadd(slot):
            bw, _, sp, _ = slot
            pltpu.make_async_copy(w_pos.at[ip_v.at[pl.ds(0, CHUNK)]], bw, sp).wait()

        def write(c, slot):
            bw, _, _, so = slot
            pltpu.async_copy(bw, out_hbm.at[pl.ds(base + c * CHUNK, CHUNK), pl.ds(0, DIM)], so)

        def wait_write(slot):
            bw, _, _, so = slot
            pltpu.make_async_copy(bw, out_hbm.at[pl.ds(base, CHUNK), pl.ds(0, DIM)], so).wait()

        issue_word(0, bufs[0])
        issue_word(1, bufs[1])
        for c in (0, 1):
            if c > 0:
                wait_posadd(bufs[c - 1])
                write(c - 1, bufs[c - 1])
            issue_word(c + 2, bufs[c + 2])
            wait_word(bufs[c])
            scale(bufs[c])
            issue_posadd(c, bufs[c])

        def group(g, carry):
            for k in range(4):
                c = 2 + 4 * g + k
                prev = bufs[(1 + k) % 4]
                nxt = bufs[k % 4]
                cur = bufs[(2 + k) % 4]
                wait_posadd(prev)
                write(c - 1, prev)
                wait_write(nxt)
                issue_word(c + 2, nxt)
                wait_word(cur)
                scale(cur)
                issue_posadd(c, cur)
            return carry

        lax.fori_loop(0, (n_chunks - 4) // 4, group, 0, unroll=False)
        for c in (n_chunks - 2, n_chunks - 1):
            prev, cur = bufs[(c - 1) % 4], bufs[c % 4]
            wait_posadd(prev)
            write(c - 1, prev)
            wait_word(cur)
            scale(cur)
            issue_posadd(c, cur)
        last = bufs[(n_chunks - 1) % 4]
        wait_posadd(last)
        write(n_chunks - 1, last)
        for b in range(4):
            wait_write(bufs[b])

    return emb


def kernel(src_word, src_pos, W_word, W_pos):
    B, L = src_word.shape
    n = B * L
    nw = 32
    iw = src_word.reshape(-1).astype(jnp.int32)
    ip = src_pos.reshape(-1).astype(jnp.int32)
    w_pos_rep = jnp.tile(W_pos, (nw, 1))
    ip = ip + (jnp.arange(n, dtype=jnp.int32) // (n // nw)) * MAXPOS
    V = W_word.shape[0]
    w_lin = _linearize_table(V, DIM)(W_word.T).reshape(V, DIM)
    out = _build(n)(w_lin, w_pos_rep, iw, ip)
    return out[:, :DIM].reshape(B, L, DIM)


# speedup vs baseline: 1.0340x; 1.0340x over previous
import functools

import jax
import jax.numpy as jnp
from jax import lax
from jax.experimental import pallas as pl
from jax.experimental.pallas import tpu as pltpu
from jax.experimental.pallas import tpu_sc as plsc

DIM = 64
SCALE = 8.0
LANES = 16
CHUNK = 256
MAXPOS = 200


VOCAB_BLOCK = 2048


@functools.lru_cache(maxsize=None)
def _linearize_table(vocab: int, dim: int):
    nb = (vocab + VOCAB_BLOCK - 1) // VOCAB_BLOCK

    def body(wt_ref, out_ref):
        t = jnp.transpose(wt_ref[...], (1, 0))
        t3 = t.reshape(VOCAB_BLOCK // 2, 2, dim)
        out_ref[...] = jnp.concatenate([t3[:, 0, :], t3[:, 1, :]], axis=1)

    return pl.pallas_call(
        body,
        grid=(nb,),
        in_specs=[pl.BlockSpec((dim, VOCAB_BLOCK), lambda i: (0, i))],
        out_specs=pl.BlockSpec((VOCAB_BLOCK // 2, 2 * dim), lambda i: (i, 0)),
        out_shape=jax.ShapeDtypeStruct((vocab // 2, 2 * dim), jnp.float32),
    )


@functools.lru_cache(maxsize=None)
def _build(n_rows: int):
    info = plsc.get_sparse_core_info()
    nw = info.num_cores * info.num_subcores
    rows_per_w = n_rows // nw
    n_chunks = rows_per_w // CHUNK
    assert rows_per_w % CHUNK == 0 and n_chunks >= 4 and (n_chunks - 4) % 4 == 0

    mesh = plsc.VectorSubcoreMesh(core_axis_name="c", subcore_axis_name="s")

    @functools.partial(
        pl.kernel,
        mesh=mesh,
        compiler_params=pltpu.CompilerParams(use_tc_tiling_on_sc=False),
        out_type=jax.ShapeDtypeStruct((n_rows, 2 * DIM), jnp.float32),
        scratch_types=(
            [pltpu.VMEM((rows_per_w,), jnp.int32)] * 2
            + [pltpu.VMEM((CHUNK, DIM), jnp.float32)] * 4
            + [pltpu.SemaphoreType.DMA] * 12
        ),
    )
    def emb(w_word, w_pos, iw_hbm, ip_hbm, out_hbm, iw_v, ip_v, *rest):
        wid = lax.axis_index("s") * info.num_cores + lax.axis_index("c")
        base = wid * rows_per_w
        pltpu.sync_copy(iw_hbm.at[pl.ds(base, rows_per_w)], iw_v)
        pltpu.sync_copy(ip_hbm.at[pl.ds(base, rows_per_w)], ip_v)

        bufv, semv = rest[:4], rest[4:]
        bufs = tuple(
            (bufv[i], semv[3 * i], semv[3 * i + 1], semv[3 * i + 2]) for i in range(4)
        )

        def issue_word(c, slot):
            bw, sw, _, _ = slot
            pltpu.async_copy(w_word.at[iw_v.at[pl.ds(c * CHUNK, CHUNK)]], bw, sw)

        def wait_word(slot):
            bw, sw, _, _ = slot
            pltpu.make_async_copy(w_word.at[iw_v.at[pl.ds(0, CHUNK)]], bw, sw).wait()

        def scale(slot):
            bw = slot[0]

            def row_body(r, carry):
                for k in range(4):
                    ri = r * 4 + k
                    for j in range(DIM // LANES):
                        s = pl.ds(j * LANES, LANES)
                        bw[ri, s] = bw[ri, s] * SCALE
                return carry

            lax.fori_loop(0, CHUNK // 4, row_body, 0, unroll=False)

        def issue_posadd(c, slot):
            bw, _, sp, _ = slot
            pltpu.async_copy(
                w_pos.at[ip_v.at[pl.ds(c * CHUNK, CHUNK)]], bw, sp, add=True
            )

        def wait_posadd(slot):
            bw, _, sp, _ = slot
            pltpu.make_async_copy(w_pos.at[ip_v.at[pl.ds(0, CHUNK)]], bw, sp).wait()

        def write(c, slot):
            bw, _, _, so = slot
            pltpu.async_copy(bw, out_hbm.at[pl.ds(base + c * CHUNK, CHUNK), pl.ds(0, DIM)], so)

        def wait_write(slot):
            bw, _, _, so = slot
            pltpu.make_async_copy(bw, out_hbm.at[pl.ds(base, CHUNK), pl.ds(0, DIM)], so).wait()

        issue_word(0, bufs[0])
        issue_word(1, bufs[1])
        for c in (0, 1):
            if c > 0:
                wait_posadd(bufs[c - 1])
                write(c - 1, bufs[c - 1])
            issue_word(c + 2, bufs[c + 2])
            wait_word(bufs[c])
            scale(bufs[c])
            issue_posadd(c, bufs[c])

        def group(g, carry):
            for k in range(4):
                c = 2 + 4 * g + k
                prev = bufs[(1 + k) % 4]
                nxt = bufs[k % 4]
                cur = bufs[(2 + k) % 4]
                wait_posadd(prev)
                write(c - 1, prev)
                wait_write(nxt)
                issue_word(c + 2, nxt)
                wait_word(cur)
                scale(cur)
                issue_posadd(c, cur)
            return carry

        lax.fori_loop(0, (n_chunks - 4) // 4, group, 0, unroll=False)
        for c in (n_chunks - 2, n_chunks - 1):
            prev, cur = bufs[(c - 1) % 4], bufs[c % 4]
            wait_posadd(prev)
            write(c - 1, prev)
            wait_word(cur)
            scale(cur)
            issue_posadd(c, cur)
        last = bufs[(n_chunks - 1) % 4]
        wait_posadd(last)
        write(n_chunks - 1, last)
        for b in range(4):
            wait_write(bufs[b])

    return emb


def kernel(src_word, src_pos, W_word, W_pos):
    B, L = src_word.shape
    n = B * L
    nw = 32
    iw = src_word.reshape(-1).astype(jnp.int32)
    ip = src_pos.reshape(-1).astype(jnp.int32)
    w_pos_rep = jnp.tile(W_pos, (nw, 1))
    ip = ip + (jnp.arange(n, dtype=jnp.int32) // (n // nw)) * MAXPOS
    V = W_word.shape[0]
    w_lin = _linearize_table(V, DIM)(W_word.T).reshape(V, DIM)
    out = _build(n)(w_lin, w_pos_rep, iw, ip)
    return out[:, :DIM].reshape(B, L, DIM)
